# baseline (speedup 1.0000x reference)
import functools

import jax
import jax.numpy as jnp
from jax import lax
from jax.experimental import pallas as pl
from jax.experimental.pallas import tpu as pltpu
from jax.experimental.pallas import tpu_sc as plsc


def _make_copy_kernel(L, D, dtype, num_cores, num_subcores):
    num_workers = num_cores * num_subcores
    rows_per_w = L // num_workers

    mesh = plsc.VectorSubcoreMesh(core_axis_name="c", subcore_axis_name="s")

    @functools.partial(
        pl.kernel,
        mesh=mesh,
        out_type=jax.ShapeDtypeStruct((L, D), dtype),
    )
    def copy_k(table_hbm, out_hbm):
        wid = lax.axis_index("s") * num_cores + lax.axis_index("c")
        base = wid * rows_per_w
        pltpu.sync_copy(
            table_hbm.at[pl.ds(base, rows_per_w)],
            out_hbm.at[pl.ds(base, rows_per_w)],
        )

    return copy_k


def kernel(hidden_states, position_table):
    L = hidden_states.shape[1]
    D = position_table.shape[1]
    copy_k = _make_copy_kernel(L, D, position_table.dtype, 2, 16)
    out = copy_k(position_table)
    return out[None]


# speedup vs baseline: 16.3224x; 16.3224x over previous
import functools

import jax
import jax.numpy as jnp
from jax import lax
from jax.experimental import pallas as pl
from jax.experimental.pallas import tpu as pltpu
from jax.experimental.pallas import tpu_sc as plsc


def _make_copy_kernel(L, D, dtype, num_cores, num_subcores):
    num_workers = num_cores * num_subcores
    rows_per_w = L // num_workers
    chunk = 32
    nchunks = rows_per_w // chunk

    mesh = plsc.VectorSubcoreMesh(core_axis_name="c", subcore_axis_name="s")

    @functools.partial(
        pl.kernel,
        mesh=mesh,
        out_type=jax.ShapeDtypeStruct((L, D), dtype),
        scratch_types=[
            pltpu.VMEM((2, chunk, D), dtype),
            pltpu.SemaphoreType.DMA,
            pltpu.SemaphoreType.DMA,
        ],
    )
    def copy_k(table_hbm, out_hbm, buf, in_sem, out_sem):
        wid = lax.axis_index("s") * num_cores + lax.axis_index("c")
        base = wid * rows_per_w

        def drain_in(slot):
            pltpu.make_async_copy(
                table_hbm.at[pl.ds(base, chunk)], buf.at[slot], in_sem
            ).wait()

        def drain_out():
            pltpu.make_async_copy(
                buf.at[0], out_hbm.at[pl.ds(base, chunk)], out_sem
            ).wait()

        pltpu.async_copy(table_hbm.at[pl.ds(base, chunk)], buf.at[0], in_sem)
        for i in range(nchunks):
            slot = i % 2
            drain_in(slot)
            pltpu.async_copy(
                buf.at[slot],
                out_hbm.at[pl.ds(base + i * chunk, chunk)],
                out_sem,
            )
            if i + 1 < nchunks:
                if i >= 1:
                    drain_out()
                pltpu.async_copy(
                    table_hbm.at[pl.ds(base + (i + 1) * chunk, chunk)],
                    buf.at[1 - slot],
                    in_sem,
                )
        drain_out()
        drain_out()

    return copy_k


def kernel(hidden_states, position_table):
    L = hidden_states.shape[1]
    D = position_table.shape[1]
    copy_k = _make_copy_kernel(L, D, position_table.dtype, 2, 16)
    out = copy_k(position_table)
    return out[None]


# speedup vs baseline: 17.0102x; 1.0421x over previous
import functools

import jax
import jax.numpy as jnp
from jax import lax
from jax.experimental import pallas as pl
from jax.experimental.pallas import tpu as pltpu
from jax.experimental.pallas import tpu_sc as plsc


def _make_copy_kernel(L, D, dtype, num_cores, num_subcores):
    num_workers = num_cores * num_subcores
    rows_per_w = L // num_workers
    chunk = 16
    nbuf = 4
    nchunks = rows_per_w // chunk

    mesh = plsc.VectorSubcoreMesh(core_axis_name="c", subcore_axis_name="s")

    @functools.partial(
        pl.kernel,
        mesh=mesh,
        out_type=jax.ShapeDtypeStruct((L, D), dtype),
        scratch_types=[
            pltpu.VMEM((nbuf, chunk, D), dtype),
            pltpu.SemaphoreType.DMA,
            pltpu.SemaphoreType.DMA,
        ],
    )
    def copy_k(table_hbm, out_hbm, buf, in_sem, out_sem):
        wid = lax.axis_index("s") * num_cores + lax.axis_index("c")
        base = wid * rows_per_w

        def load(j):
            pltpu.async_copy(
                table_hbm.at[pl.ds(base + j * chunk, chunk)],
                buf.at[j % nbuf],
                in_sem,
            )

        def store(j):
            pltpu.async_copy(
                buf.at[j % nbuf],
                out_hbm.at[pl.ds(base + j * chunk, chunk)],
                out_sem,
            )

        def drain_in(j):
            pltpu.make_async_copy(
                table_hbm.at[pl.ds(base, chunk)], buf.at[j % nbuf], in_sem
            ).wait()

        def drain_out(j):
            pltpu.make_async_copy(
                buf.at[j % nbuf], out_hbm.at[pl.ds(base, chunk)], out_sem
            ).wait()

        for j in range(min(nbuf - 1, nchunks)):
            load(j)
        for i in range(nchunks):
            if i >= 1:
                drain_out(i - 1)
            j = i + nbuf - 1
            if j < nchunks:
                load(j)
            drain_in(i)
            store(i)
        drain_out(nchunks - 1)

    return copy_k


def kernel(hidden_states, position_table):
    L = hidden_states.shape[1]
    D = position_table.shape[1]
    copy_k = _make_copy_kernel(L, D, position_table.dtype, 2, 16)
    out = copy_k(position_table)
    return out[None]


# speedup vs baseline: 17.2173x; 1.0122x over previous
import functools

import jax
import jax.numpy as jnp
from jax import lax
from jax.experimental import pallas as pl
from jax.experimental.pallas import tpu as pltpu
from jax.experimental.pallas import tpu_sc as plsc


def _make_copy_kernel(L, D, dtype, num_cores, num_subcores):
    num_workers = num_cores * num_subcores
    rows_per_w = L // num_workers
    chunk = 32
    nbuf = 3
    nchunks = rows_per_w // chunk

    mesh = plsc.VectorSubcoreMesh(core_axis_name="c", subcore_axis_name="s")

    @functools.partial(
        pl.kernel,
        mesh=mesh,
        out_type=jax.ShapeDtypeStruct((L, D), dtype),
        scratch_types=[
            pltpu.VMEM((nbuf, chunk, D), dtype),
            pltpu.SemaphoreType.DMA,
            pltpu.SemaphoreType.DMA,
        ],
    )
    def copy_k(table_hbm, out_hbm, buf, in_sem, out_sem):
        wid = lax.axis_index("s") * num_cores + lax.axis_index("c")
        base = wid * rows_per_w

        def load(j):
            pltpu.async_copy(
                table_hbm.at[pl.ds(base + j * chunk, chunk)],
                buf.at[j % nbuf],
                in_sem,
            )

        def store(j):
            pltpu.async_copy(
                buf.at[j % nbuf],
                out_hbm.at[pl.ds(base + j * chunk, chunk)],
                out_sem,
            )

        def drain_in(j):
            pltpu.make_async_copy(
                table_hbm.at[pl.ds(base, chunk)], buf.at[j % nbuf], in_sem
            ).wait()

        def drain_out(j):
            pltpu.make_async_copy(
                buf.at[j % nbuf], out_hbm.at[pl.ds(base, chunk)], out_sem
            ).wait()

        for j in range(min(nbuf - 1, nchunks)):
            load(j)
        for i in range(nchunks):
            if i >= 1:
                drain_out(i - 1)
            j = i + nbuf - 1
            if j < nchunks:
                load(j)
            drain_in(i)
            store(i)
        drain_out(nchunks - 1)

    return copy_k


def kernel(hidden_states, position_table):
    L = hidden_states.shape[1]
    D = position_table.shape[1]
    copy_k = _make_copy_kernel(L, D, position_table.dtype, 2, 16)
    out = copy_k(position_table)
    return out[None]
